# baseline (speedup 1.0000x reference)
import functools

import jax
import jax.numpy as jnp
from jax import lax
from jax.experimental import pallas as pl
from jax.experimental.pallas import tpu as pltpu
from jax.experimental.pallas import tpu_sc as plsc

BATCH = 16384
D = 64
NC = 2
NS = 16
NW = NC * NS
B_PER_W = BATCH // NW


CHUNK = 128
N_ROWS = 100000


def _sc_gather(big_table, users, movies, nbatch):
  mesh = plsc.VectorSubcoreMesh(core_axis_name="c", subcore_axis_name="s")
  b_per_w = nbatch // NW
  nch = b_per_w // CHUNK
  cp = pltpu.CompilerParams(
      skip_device_barrier=True,
      disable_semaphore_checks=True,
      disable_bounds_checks=True,
  )

  @functools.partial(
      pl.kernel,
      mesh=mesh,
      out_type=[
          jax.ShapeDtypeStruct((nbatch, 2 * D), jnp.float32),
          jax.ShapeDtypeStruct((nbatch, 2 * D), jnp.float32),
      ],
      scratch_types=[
          pltpu.VMEM((b_per_w,), jnp.int32),
          pltpu.VMEM((b_per_w,), jnp.int32),
          pltpu.VMEM((CHUNK, 2 * D), jnp.float32),
          pltpu.VMEM((CHUNK, 2 * D), jnp.float32),
          pltpu.VMEM((CHUNK, 2 * D), jnp.float32),
          pltpu.VMEM((CHUNK, 2 * D), jnp.float32),
          [pltpu.SemaphoreType.DMA] * 10,
      ],
      compiler_params=cp,
  )
  def gather_kernel(table_hbm, users_hbm, movies_hbm, ou_hbm, om_hbm,
                    uidx_v, midx_v, ubuf0, ubuf1, mbuf0, mbuf1, sems):
    wid = lax.axis_index("s") * NC + lax.axis_index("c")
    base = wid * b_per_w
    ubufs = (ubuf0, ubuf1)
    mbufs = (mbuf0, mbuf1)
    hu = pltpu.async_copy(users_hbm.at[pl.ds(base, b_per_w)], uidx_v, sems[8])
    hm = pltpu.async_copy(movies_hbm.at[pl.ds(base, b_per_w)], midx_v, sems[9])
    hu.wait()
    hm.wait()

    gu = [None] * nch
    gm = [None] * nch
    wu = [None] * nch
    wm = [None] * nch

    def issue_gather(k):
      p = k % 2
      gu[k] = pltpu.async_copy(
          table_hbm.at[uidx_v.at[pl.ds(k * CHUNK, CHUNK)]], ubufs[p], sems[p])
      gm[k] = pltpu.async_copy(
          table_hbm.at[midx_v.at[pl.ds(k * CHUNK, CHUNK)]], mbufs[p],
          sems[2 + p])

    issue_gather(0)
    for k in range(nch):
      p = k % 2
      if k + 1 < nch:
        if k >= 1:
          wu[k - 1].wait()
          wm[k - 1].wait()
        issue_gather(k + 1)
      gu[k].wait()
      gm[k].wait()
      wu[k] = pltpu.async_copy(
          ubufs[p], ou_hbm.at[pl.ds(base + k * CHUNK, CHUNK)], sems[4 + p])
      wm[k] = pltpu.async_copy(
          mbufs[p], om_hbm.at[pl.ds(base + k * CHUNK, CHUNK)], sems[6 + p])
    wu[nch - 2].wait()
    wm[nch - 2].wait()
    wu[nch - 1].wait()
    wm[nch - 1].wait()

  return gather_kernel(big_table, users, movies)


def _mlp_body(u_ref, m_ref, w1a_ref, w1b_ref, w1c_ref, b1_ref, w2_ref, b2_ref,
              o_ref):
  u = u_ref[:, :D]
  m = m_ref[:, D:]
  h = (
      jnp.dot(u * m, w1a_ref[...], preferred_element_type=jnp.float32)
      + jnp.dot(u, w1b_ref[...], preferred_element_type=jnp.float32)
      + jnp.dot(m, w1c_ref[...], preferred_element_type=jnp.float32)
      + b1_ref[...]
  )
  h = jnp.maximum(h, 0.0)
  y = jnp.dot(h, w2_ref[...], preferred_element_type=jnp.float32) + b2_ref[...]
  o_ref[...] = jax.nn.sigmoid(y)


def _tc_mlp(u_g, m_g, W1, b1, W2, b2, nbatch, block=2048):
  w1t = W1.T
  w1a = w1t[:D]
  w1b = w1t[D:2 * D]
  w1c = w1t[2 * D:]
  b1r = b1.reshape(1, 8)
  w2r = W2.reshape(8, 1)
  b2r = b2.reshape(1, 1)
  grid = (nbatch // block,)
  return pl.pallas_call(
      _mlp_body,
      grid=grid,
      in_specs=[
          pl.BlockSpec((block, 2 * D), lambda i: (i, 0)),
          pl.BlockSpec((block, 2 * D), lambda i: (i, 0)),
          pl.BlockSpec((D, 8), lambda i: (0, 0)),
          pl.BlockSpec((D, 8), lambda i: (0, 0)),
          pl.BlockSpec((D, 8), lambda i: (0, 0)),
          pl.BlockSpec((1, 8), lambda i: (0, 0)),
          pl.BlockSpec((8, 1), lambda i: (0, 0)),
          pl.BlockSpec((1, 1), lambda i: (0, 0)),
      ],
      out_specs=pl.BlockSpec((block, 1), lambda i: (i, 0)),
      out_shape=jax.ShapeDtypeStruct((nbatch, 1), jnp.float32),
  )(u_g, m_g, w1a, w1b, w1c, b1r, w2r, b2r)


@jax.jit
def kernel(users, movies, user_emb, movie_emb, W1, b1, W2, b2):
  users = users.astype(jnp.int32)
  movies = movies.astype(jnp.int32)
  big_table = jnp.concatenate([user_emb, movie_emb], axis=1)
  u_g, m_g = _sc_gather(big_table, users, movies, BATCH)
  return _tc_mlp(u_g, m_g, W1, b1, W2, b2, BATCH)


# speedup vs baseline: 1.0145x; 1.0145x over previous
import functools

import jax
import jax.numpy as jnp
from jax import lax
from jax.experimental import pallas as pl
from jax.experimental.pallas import tpu as pltpu
from jax.experimental.pallas import tpu_sc as plsc

BATCH = 16384
D = 64
NC = 2
NS = 16
NW = NC * NS
B_PER_W = BATCH // NW


CHUNK = 128
N_ROWS = 100000


def _sc_gather(big_table, users, movies, nbatch):
  mesh = plsc.VectorSubcoreMesh(core_axis_name="c", subcore_axis_name="s")
  b_per_w = nbatch // NW
  nch = b_per_w // CHUNK
  cp = pltpu.CompilerParams(
      skip_device_barrier=True,
      disable_semaphore_checks=True,
      disable_bounds_checks=True,
  )

  @functools.partial(
      pl.kernel,
      mesh=mesh,
      out_type=[
          jax.ShapeDtypeStruct((nbatch, 2 * D), jnp.float32),
          jax.ShapeDtypeStruct((nbatch, 2 * D), jnp.float32),
      ],
      scratch_types=[
          pltpu.VMEM((b_per_w,), jnp.int32),
          pltpu.VMEM((b_per_w,), jnp.int32),
          pltpu.VMEM((CHUNK, 2 * D), jnp.float32),
          pltpu.VMEM((CHUNK, 2 * D), jnp.float32),
          pltpu.VMEM((CHUNK, 2 * D), jnp.float32),
          pltpu.VMEM((CHUNK, 2 * D), jnp.float32),
          [pltpu.SemaphoreType.DMA] * 10,
      ],
      compiler_params=cp,
  )
  def gather_kernel(table_hbm, users_hbm, movies_hbm, ou_hbm, om_hbm,
                    uidx_v, midx_v, ubuf0, ubuf1, mbuf0, mbuf1, sems):
    wid = lax.axis_index("s") * NC + lax.axis_index("c")
    base = wid * b_per_w
    ubufs = (ubuf0, ubuf1)
    mbufs = (mbuf0, mbuf1)
    hu = pltpu.async_copy(users_hbm.at[pl.ds(base, b_per_w)], uidx_v, sems[8])
    hm = pltpu.async_copy(movies_hbm.at[pl.ds(base, b_per_w)], midx_v, sems[9])
    hu.wait()
    hm.wait()

    gu = [None] * nch
    gm = [None] * nch
    wu = [None] * nch
    wm = [None] * nch

    def issue_gather(k):
      p = k % 2
      gu[k] = pltpu.async_copy(
          table_hbm.at[uidx_v.at[pl.ds(k * CHUNK, CHUNK)]], ubufs[p], sems[p])
      gm[k] = pltpu.async_copy(
          table_hbm.at[midx_v.at[pl.ds(k * CHUNK, CHUNK)]], mbufs[p],
          sems[2 + p])

    issue_gather(0)
    for k in range(nch):
      p = k % 2
      if k + 1 < nch:
        if k >= 1:
          wu[k - 1].wait()
          wm[k - 1].wait()
        issue_gather(k + 1)
      gu[k].wait()
      gm[k].wait()
      wu[k] = pltpu.async_copy(
          ubufs[p], ou_hbm.at[pl.ds(base + k * CHUNK, CHUNK)], sems[4 + p])
      wm[k] = pltpu.async_copy(
          mbufs[p], om_hbm.at[pl.ds(base + k * CHUNK, CHUNK)], sems[6 + p])
    wu[nch - 2].wait()
    wm[nch - 2].wait()
    wu[nch - 1].wait()
    wm[nch - 1].wait()

  return gather_kernel(big_table, users, movies)


def _mlp_body(u_ref, m_ref, w1a_ref, w1b_ref, w1c_ref, b1_ref, w2_ref, b2_ref,
              o_ref):
  u = u_ref[:, :D]
  m = m_ref[:, D:]
  h = (
      jnp.dot(u * m, w1a_ref[...], preferred_element_type=jnp.float32)
      + jnp.dot(u, w1b_ref[...], preferred_element_type=jnp.float32)
      + jnp.dot(m, w1c_ref[...], preferred_element_type=jnp.float32)
      + b1_ref[...]
  )
  h = jnp.maximum(h, 0.0)
  y = jnp.dot(h, w2_ref[...], preferred_element_type=jnp.float32) + b2_ref[...]
  o_ref[...] = jax.nn.sigmoid(y)


def _tc_mlp(u_g, m_g, W1, b1, W2, b2, nbatch, block=2048):
  w1t = W1.T
  w1a = w1t[:D]
  w1b = w1t[D:2 * D]
  w1c = w1t[2 * D:]
  b1r = b1.reshape(1, 8)
  w2r = W2.reshape(8, 1)
  b2r = b2.reshape(1, 1)
  grid = (nbatch // block,)
  return pl.pallas_call(
      _mlp_body,
      grid=grid,
      in_specs=[
          pl.BlockSpec((block, 2 * D), lambda i: (i, 0)),
          pl.BlockSpec((block, 2 * D), lambda i: (i, 0)),
          pl.BlockSpec((D, 8), lambda i: (0, 0)),
          pl.BlockSpec((D, 8), lambda i: (0, 0)),
          pl.BlockSpec((D, 8), lambda i: (0, 0)),
          pl.BlockSpec((1, 8), lambda i: (0, 0)),
          pl.BlockSpec((8, 1), lambda i: (0, 0)),
          pl.BlockSpec((1, 1), lambda i: (0, 0)),
      ],
      out_specs=pl.BlockSpec((block, 1), lambda i: (i, 0)),
      out_shape=jax.ShapeDtypeStruct((nbatch, 1), jnp.float32),
  )(u_g, m_g, w1a, w1b, w1c, b1r, w2r, b2r)


@jax.jit
def kernel(users, movies, user_emb, movie_emb, W1, b1, W2, b2):
  users = users.astype(jnp.int32)
  movies = movies.astype(jnp.int32)
  big_table = jnp.concatenate([user_emb, movie_emb], axis=1)
  npipe = 2
  piece = BATCH // npipe
  outs = []
  for p in range(npipe):
    sl = slice(p * piece, (p + 1) * piece)
    u_g, m_g = _sc_gather(big_table, users[sl], movies[sl], piece)
    outs.append(_tc_mlp(u_g, m_g, W1, b1, W2, b2, piece))
  return jnp.concatenate(outs, axis=0)
